# Initial kernel scaffold; baseline (speedup 1.0000x reference)
import jax
import jax.numpy as jnp
from jax.experimental import pallas as pl


def kernel(x, Wg, bg, W1, b1, W2, b2):
    raise NotImplementedError("write your pallas kernel here")


# speedup vs baseline: 2.0113x; 2.0113x over previous
import functools

import jax
import jax.numpy as jnp
import numpy as np
from jax import lax
from jax.experimental import pallas as pl
from jax.experimental.pallas import tpu as pltpu

N = 4096
DM = 1024
DH = 4096
E = 16
BT = 128
NTILE = (8192 + E * BT) // BT
NP = NTILE * BT
NB = N // BT
EPS = float(np.finfo(float).eps)


def _gate_route_body(x_ref, wg_ref, bg_ref, e1_ref, e2_ref, g1_ref, g2_ref,
                     r1_ref, r2_ref, cnt_ref, run_ref):
    t = pl.program_id(0)

    @pl.when(t == 0)
    def _init():
        run_ref[...] = jnp.zeros_like(run_ref)

    x = x_ref[...]
    logits = lax.dot_general(x, wg_ref[...], (((1,), (1,)), ((), ()))) + bg_ref[...]
    iota = lax.broadcasted_iota(jnp.int32, (BT, E), 1).astype(jnp.float32)
    m1 = jnp.max(logits, axis=1, keepdims=True)
    e1 = jnp.min(jnp.where(logits == m1, iota, float(E)), axis=1, keepdims=True)
    masked = jnp.where(iota == e1, -jnp.inf, logits)
    m2 = jnp.max(masked, axis=1, keepdims=True)
    e2 = jnp.min(jnp.where(masked == m2, iota, float(E)), axis=1, keepdims=True)
    g1 = 1.0 / (1.0 + jnp.exp(m2 - m1))
    g2 = 1.0 - g1

    O1 = (iota == e1).astype(jnp.float32)
    O2 = (iota == e2).astype(jnp.float32)
    C = O1 + O2
    r_i = lax.broadcasted_iota(jnp.int32, (BT, BT), 0)
    c_i = lax.broadcasted_iota(jnp.int32, (BT, BT), 1)
    tri = (c_i < r_i).astype(jnp.float32)
    cex = lax.dot_general(tri, C, (((1,), (0,)), ((), ()))) + run_ref[...]
    r1 = jnp.sum(O1 * cex, axis=1)
    r2 = jnp.sum(O2 * cex, axis=1)

    newrun = run_ref[...] + jnp.sum(C, axis=0, keepdims=True)
    run_ref[...] = newrun
    cnt_ref[...] = newrun

    e1_ref[...] = e1.astype(jnp.int32).reshape(1, 1, BT)
    e2_ref[...] = e2.astype(jnp.int32).reshape(1, 1, BT)
    g1_ref[...] = g1.reshape(1, 1, BT)
    g2_ref[...] = g2.reshape(1, 1, BT)
    r1_ref[...] = r1.astype(jnp.int32).reshape(1, 1, BT)
    r2_ref[...] = r2.astype(jnp.int32).reshape(1, 1, BT)


def _gate_route(x, Wg, bg):
    i32b = jax.ShapeDtypeStruct((NB, 1, BT), jnp.int32)
    f32b = jax.ShapeDtypeStruct((NB, 1, BT), jnp.float32)
    outs = (i32b, i32b, f32b, f32b, i32b, i32b,
            jax.ShapeDtypeStruct((1, E), jnp.float32))
    blk3 = pl.BlockSpec((1, 1, BT), lambda t: (t, 0, 0))
    return pl.pallas_call(
        _gate_route_body,
        grid=(NB,),
        in_specs=[
            pl.BlockSpec((BT, DM), lambda t: (t, 0)),
            pl.BlockSpec((E, DM), lambda t: (0, 0)),
            pl.BlockSpec((1, E), lambda t: (0, 0)),
        ],
        out_specs=(blk3, blk3, blk3, blk3, blk3, blk3,
                   pl.BlockSpec((1, E), lambda t: (0, 0))),
        out_shape=outs,
        scratch_shapes=[pltpu.VMEM((1, E), jnp.float32)],
        compiler_params=pltpu.CompilerParams(
            dimension_semantics=("arbitrary",)),
    )(x, Wg, bg.reshape(1, E))


def _make_dest_body(cnt_ref, e1_ref, e2_ref, r1_ref, r2_ref,
                    d1_ref, d2_ref, te_ref):
    counts = cnt_ref[...]
    ccnt = jnp.ceil(counts / BT) * BT
    e_i = lax.broadcasted_iota(jnp.int32, (E, E), 0)
    e_j = lax.broadcasted_iota(jnp.int32, (E, E), 1)
    tri = (e_j < e_i).astype(jnp.float32)
    offs = lax.dot_general(ccnt, tri, (((1,), (1,)), ((), ())))

    iota = lax.broadcasted_iota(jnp.int32, (BT, E), 1)
    e1 = e1_ref[...].reshape(BT, 1)
    e2 = e2_ref[...].reshape(BT, 1)
    O1 = (iota == e1).astype(jnp.float32)
    O2 = (iota == e2).astype(jnp.float32)
    b1 = jnp.sum(O1 * offs, axis=1).astype(jnp.int32)
    b2 = jnp.sum(O2 * offs, axis=1).astype(jnp.int32)
    d1_ref[...] = (b1 + r1_ref[...].reshape(BT)).reshape(1, 1, BT)
    d2_ref[...] = (b2 + r2_ref[...].reshape(BT)).reshape(1, 1, BT)

    ends = offs + ccnt
    tid = lax.broadcasted_iota(jnp.int32, (BT, E), 0).astype(jnp.float32) * float(BT)
    full = jnp.sum((tid >= ends).astype(jnp.int32), axis=1)
    te_ref[...] = jnp.minimum(full, E - 1).reshape(1, BT)


def _make_dest(counts, e1, e2, r1, r2):
    i32b = jax.ShapeDtypeStruct((NB, 1, BT), jnp.int32)
    blk3 = pl.BlockSpec((1, 1, BT), lambda t: (t, 0, 0))
    return pl.pallas_call(
        _make_dest_body,
        grid=(NB,),
        in_specs=[pl.BlockSpec((1, E), lambda t: (0, 0)),
                  blk3, blk3, blk3, blk3],
        out_specs=(blk3, blk3, pl.BlockSpec((1, BT), lambda t: (0, 0))),
        out_shape=(i32b, i32b, jax.ShapeDtypeStruct((1, BT), jnp.int32)),
        compiler_params=pltpu.CompilerParams(
            dimension_semantics=("arbitrary",)),
    )(counts, e1, e2, r1, r2)


def _layer1_body(te_ref, xg_ref, w1_ref, b1_ref, hg_ref):
    h = lax.dot_general(xg_ref[...], w1_ref[0], (((1,), (1,)), ((), ())))
    hg_ref[...] = jnp.maximum(h + b1_ref[0], 0.0)


def _layer2_body(te_ref, hg_ref, w2_ref, b2_ref, yg_ref):
    y = lax.dot_general(hg_ref[...], w2_ref[0], (((1,), (1,)), ((), ())))
    yg_ref[...] = jnp.exp(jnp.maximum(y + b2_ref[0], 0.0))


def _expert_mlp(tile_expert, xg, W1, b1, W2, b2):
    gs1 = pltpu.PrefetchScalarGridSpec(
        num_scalar_prefetch=1,
        grid=(NTILE,),
        in_specs=[
            pl.BlockSpec((BT, DM), lambda t, te: (t, 0)),
            pl.BlockSpec((1, DH, DM), lambda t, te: (te[t], 0, 0)),
            pl.BlockSpec((1, 1, DH), lambda t, te: (te[t], 0, 0)),
        ],
        out_specs=pl.BlockSpec((BT, DH), lambda t, te: (t, 0)),
    )
    hg = pl.pallas_call(
        _layer1_body,
        grid_spec=gs1,
        out_shape=jax.ShapeDtypeStruct((NP, DH), jnp.float32),
        compiler_params=pltpu.CompilerParams(
            dimension_semantics=("arbitrary",)),
    )(tile_expert, xg, W1, b1.reshape(E, 1, DH))
    gs2 = pltpu.PrefetchScalarGridSpec(
        num_scalar_prefetch=1,
        grid=(NTILE,),
        in_specs=[
            pl.BlockSpec((BT, DH), lambda t, te: (t, 0)),
            pl.BlockSpec((1, DM, DH), lambda t, te: (te[t], 0, 0)),
            pl.BlockSpec((1, 1, DM), lambda t, te: (te[t], 0, 0)),
        ],
        out_specs=pl.BlockSpec((BT, DM), lambda t, te: (t, 0)),
    )
    return pl.pallas_call(
        _layer2_body,
        grid_spec=gs2,
        out_shape=jax.ShapeDtypeStruct((NP, DM), jnp.float32),
        compiler_params=pltpu.CompilerParams(
            dimension_semantics=("arbitrary",)),
    )(tile_expert, hg, W2, b2.reshape(E, 1, DM))


def _log_body(s_ref, o_ref):
    o_ref[...] = jnp.log(s_ref[...] + EPS)


def _log_eps(S):
    return pl.pallas_call(
        _log_body,
        grid=(NB,),
        in_specs=[pl.BlockSpec((BT, DM), lambda t: (t, 0))],
        out_specs=pl.BlockSpec((BT, DM), lambda t: (t, 0)),
        out_shape=jax.ShapeDtypeStruct((N, DM), jnp.float32),
    )(S)


def kernel(x, Wg, bg, W1, b1, W2, b2):
    e1, e2, g1, g2, r1, r2, counts = _gate_route(x, Wg, bg)
    d1, d2, te = _make_dest(counts, e1, e2, r1, r2)
    dest1 = d1.reshape(N)
    dest2 = d2.reshape(N)
    tile_expert = te.reshape(BT)[:NTILE]

    xg = jnp.zeros((NP, DM), jnp.float32).at[dest1].set(x).at[dest2].set(x)

    yg = _expert_mlp(tile_expert, xg, W1, b1, W2, b2)

    S = (g1.reshape(N, 1) * yg[dest1] + g2.reshape(N, 1) * yg[dest2])
    return _log_eps(S)


# speedup vs baseline: 2.0687x; 1.0286x over previous
import functools

import jax
import jax.numpy as jnp
import numpy as np
from jax import lax
from jax.experimental import pallas as pl
from jax.experimental.pallas import tpu as pltpu
from jax.experimental.pallas import tpu_sc as plsc

N = 4096
DM = 1024
DH = 4096
E = 16
BT = 128
NTILE = (8192 + E * BT) // BT
NP = NTILE * BT
NB = N // BT
EPS = float(np.finfo(float).eps)


def _gate_route_body(x_ref, wg_ref, bg_ref, e1_ref, e2_ref, g1_ref, g2_ref,
                     r1_ref, r2_ref, cnt_ref, run_ref):
    t = pl.program_id(0)

    @pl.when(t == 0)
    def _init():
        run_ref[...] = jnp.zeros_like(run_ref)

    x = x_ref[...]
    logits = lax.dot_general(x, wg_ref[...], (((1,), (1,)), ((), ()))) + bg_ref[...]
    iota = lax.broadcasted_iota(jnp.int32, (BT, E), 1).astype(jnp.float32)
    m1 = jnp.max(logits, axis=1, keepdims=True)
    e1 = jnp.min(jnp.where(logits == m1, iota, float(E)), axis=1, keepdims=True)
    masked = jnp.where(iota == e1, -jnp.inf, logits)
    m2 = jnp.max(masked, axis=1, keepdims=True)
    e2 = jnp.min(jnp.where(masked == m2, iota, float(E)), axis=1, keepdims=True)
    g1 = 1.0 / (1.0 + jnp.exp(m2 - m1))
    g2 = 1.0 - g1

    O1 = (iota == e1).astype(jnp.float32)
    O2 = (iota == e2).astype(jnp.float32)
    C = O1 + O2
    r_i = lax.broadcasted_iota(jnp.int32, (BT, BT), 0)
    c_i = lax.broadcasted_iota(jnp.int32, (BT, BT), 1)
    tri = (c_i < r_i).astype(jnp.float32)
    cex = lax.dot_general(tri, C, (((1,), (0,)), ((), ()))) + run_ref[...]
    r1 = jnp.sum(O1 * cex, axis=1)
    r2 = jnp.sum(O2 * cex, axis=1)

    newrun = run_ref[...] + jnp.sum(C, axis=0, keepdims=True)
    run_ref[...] = newrun
    cnt_ref[...] = newrun

    e1_ref[...] = e1.astype(jnp.int32).reshape(1, 1, BT)
    e2_ref[...] = e2.astype(jnp.int32).reshape(1, 1, BT)
    g1_ref[...] = jnp.broadcast_to(g1, (BT, 16)).reshape(1, BT, 16)
    g2_ref[...] = jnp.broadcast_to(g2, (BT, 16)).reshape(1, BT, 16)
    r1_ref[...] = r1.astype(jnp.int32).reshape(1, 1, BT)
    r2_ref[...] = r2.astype(jnp.int32).reshape(1, 1, BT)


def _gate_route(x, Wg, bg):
    i32b = jax.ShapeDtypeStruct((NB, 1, BT), jnp.int32)
    f32x = jax.ShapeDtypeStruct((NB, BT, 16), jnp.float32)
    outs = (i32b, i32b, f32x, f32x, i32b, i32b,
            jax.ShapeDtypeStruct((1, E), jnp.float32))
    blk3 = pl.BlockSpec((1, 1, BT), lambda t: (t, 0, 0))
    blkx = pl.BlockSpec((1, BT, 16), lambda t: (t, 0, 0))
    return pl.pallas_call(
        _gate_route_body,
        grid=(NB,),
        in_specs=[
            pl.BlockSpec((BT, DM), lambda t: (t, 0)),
            pl.BlockSpec((E, DM), lambda t: (0, 0)),
            pl.BlockSpec((1, E), lambda t: (0, 0)),
        ],
        out_specs=(blk3, blk3, blkx, blkx, blk3, blk3,
                   pl.BlockSpec((1, E), lambda t: (0, 0))),
        out_shape=outs,
        scratch_shapes=[pltpu.VMEM((1, E), jnp.float32)],
        compiler_params=pltpu.CompilerParams(
            dimension_semantics=("arbitrary",)),
    )(x, Wg, bg.reshape(1, E))


def _make_dest_body(cnt_ref, e1_ref, e2_ref, r1_ref, r2_ref,
                    d1_ref, d2_ref, te_ref):
    counts = cnt_ref[...]
    ccnt = jnp.ceil(counts / BT) * BT
    e_i = lax.broadcasted_iota(jnp.int32, (E, E), 0)
    e_j = lax.broadcasted_iota(jnp.int32, (E, E), 1)
    tri = (e_j < e_i).astype(jnp.float32)
    offs = lax.dot_general(ccnt, tri, (((1,), (1,)), ((), ())))

    iota = lax.broadcasted_iota(jnp.int32, (BT, E), 1)
    e1 = e1_ref[...].reshape(BT, 1)
    e2 = e2_ref[...].reshape(BT, 1)
    O1 = (iota == e1).astype(jnp.float32)
    O2 = (iota == e2).astype(jnp.float32)
    b1 = jnp.sum(O1 * offs, axis=1).astype(jnp.int32)
    b2 = jnp.sum(O2 * offs, axis=1).astype(jnp.int32)
    d1_ref[...] = (b1 + r1_ref[...].reshape(BT)).reshape(1, 1, BT)
    d2_ref[...] = (b2 + r2_ref[...].reshape(BT)).reshape(1, 1, BT)

    ends = offs + ccnt
    tid = lax.broadcasted_iota(jnp.int32, (BT, E), 0).astype(jnp.float32) * float(BT)
    full = jnp.sum((tid >= ends).astype(jnp.int32), axis=1)
    te_ref[...] = jnp.minimum(full, E - 1).reshape(1, BT)


def _make_dest(counts, e1, e2, r1, r2):
    i32b = jax.ShapeDtypeStruct((NB, 1, BT), jnp.int32)
    blk3 = pl.BlockSpec((1, 1, BT), lambda t: (t, 0, 0))
    return pl.pallas_call(
        _make_dest_body,
        grid=(NB,),
        in_specs=[pl.BlockSpec((1, E), lambda t: (0, 0)),
                  blk3, blk3, blk3, blk3],
        out_specs=(blk3, blk3, pl.BlockSpec((1, BT), lambda t: (0, 0))),
        out_shape=(i32b, i32b, jax.ShapeDtypeStruct((1, BT), jnp.int32)),
        compiler_params=pltpu.CompilerParams(
            dimension_semantics=("arbitrary",)),
    )(counts, e1, e2, r1, r2)


def _layer1_body(te_ref, xg_ref, w1_ref, b1_ref, hg_ref):
    h = lax.dot_general(xg_ref[...], w1_ref[0], (((1,), (1,)), ((), ())))
    hg_ref[...] = jnp.maximum(h + b1_ref[0], 0.0)


def _layer2_body(te_ref, hg_ref, w2_ref, b2_ref, yg_ref):
    y = lax.dot_general(hg_ref[...], w2_ref[0], (((1,), (1,)), ((), ())))
    yg_ref[...] = jnp.exp(jnp.maximum(y + b2_ref[0], 0.0))


def _expert_mlp(tile_expert, xg, W1, b1, W2, b2):
    gs1 = pltpu.PrefetchScalarGridSpec(
        num_scalar_prefetch=1,
        grid=(NTILE,),
        in_specs=[
            pl.BlockSpec((BT, DM), lambda t, te: (t, 0)),
            pl.BlockSpec((1, DH, DM), lambda t, te: (te[t], 0, 0)),
            pl.BlockSpec((1, 1, DH), lambda t, te: (te[t], 0, 0)),
        ],
        out_specs=pl.BlockSpec((BT, DH), lambda t, te: (t, 0)),
    )
    hg = pl.pallas_call(
        _layer1_body,
        grid_spec=gs1,
        out_shape=jax.ShapeDtypeStruct((NP, DH), jnp.float32),
        compiler_params=pltpu.CompilerParams(
            dimension_semantics=("arbitrary",)),
    )(tile_expert, xg, W1, b1.reshape(E, 1, DH))
    gs2 = pltpu.PrefetchScalarGridSpec(
        num_scalar_prefetch=1,
        grid=(NTILE,),
        in_specs=[
            pl.BlockSpec((BT, DH), lambda t, te: (t, 0)),
            pl.BlockSpec((1, DM, DH), lambda t, te: (te[t], 0, 0)),
            pl.BlockSpec((1, 1, DM), lambda t, te: (te[t], 0, 0)),
        ],
        out_specs=pl.BlockSpec((BT, DM), lambda t, te: (t, 0)),
    )
    return pl.pallas_call(
        _layer2_body,
        grid_spec=gs2,
        out_shape=jax.ShapeDtypeStruct((NP, DM), jnp.float32),
        compiler_params=pltpu.CompilerParams(
            dimension_semantics=("arbitrary",)),
    )(tile_expert, hg, W2, b2.reshape(E, 1, DM))


_SC_MESH = plsc.VectorSubcoreMesh(core_axis_name="c", subcore_axis_name="s")
_NC = _SC_MESH.num_cores
_NS = _SC_MESH.num_subcores
_NW = _NC * _NS
TPW = N // _NW
CH = 32


def _dispatch_body(x_hbm, d1_hbm, d2_hbm, xg_hbm, buf, idx1, idx2, sem1, sem2):
    wid = lax.axis_index("s") * _NC + lax.axis_index("c")
    base = pl.multiple_of(wid * TPW, TPW)
    for c in range(TPW // CH):
        off = base + c * CH
        pltpu.sync_copy(x_hbm.at[pl.ds(off, CH)], buf)
        pltpu.sync_copy(d1_hbm.at[pl.ds(off, CH)], idx1)
        pltpu.sync_copy(d2_hbm.at[pl.ds(off, CH)], idx2)
        cp1 = pltpu.async_copy(buf, xg_hbm.at[idx1], sem1)
        cp2 = pltpu.async_copy(buf, xg_hbm.at[idx2], sem2)
        cp1.wait()
        cp2.wait()


def _dispatch(x, dest1, dest2):
    return pl.kernel(
        _dispatch_body,
        out_type=jax.ShapeDtypeStruct((NP, DM), jnp.float32),
        mesh=_SC_MESH,
        scratch_types=[
            pltpu.VMEM((CH, DM), jnp.float32),
            pltpu.VMEM((CH,), jnp.int32),
            pltpu.VMEM((CH,), jnp.int32),
            pltpu.SemaphoreType.DMA,
            pltpu.SemaphoreType.DMA,
        ],
    )(x, dest1, dest2)


def _combine_body(yg_hbm, d1_hbm, d2_hbm, g1_hbm, g2_hbm, s_hbm,
                  buf0, buf1, g1v, g2v, idx1, idx2, sem1, sem2):
    wid = lax.axis_index("s") * _NC + lax.axis_index("c")
    base = pl.multiple_of(wid * TPW, TPW)
    for c in range(TPW // CH):
        off = base + c * CH
        pltpu.sync_copy(d1_hbm.at[pl.ds(off, CH)], idx1)
        pltpu.sync_copy(d2_hbm.at[pl.ds(off, CH)], idx2)
        pltpu.sync_copy(g1_hbm.at[pl.ds(off, CH)], g1v)
        pltpu.sync_copy(g2_hbm.at[pl.ds(off, CH)], g2v)
        cp1 = pltpu.async_copy(yg_hbm.at[idx1], buf0, sem1)
        cp2 = pltpu.async_copy(yg_hbm.at[idx2], buf1, sem2)
        cp1.wait()
        cp2.wait()

        def tok(i, _):
            s1 = g1v[i, :]
            s2 = g2v[i, :]

            def vec(v, _):
                sl = pl.ds(v * 16, 16)
                buf0[i, sl] = s1 * buf0[i, sl] + s2 * buf1[i, sl]
                return 0

            lax.fori_loop(0, DM // 16, vec, 0, unroll=4)
            return 0

        lax.fori_loop(0, CH, tok, 0)
        pltpu.sync_copy(buf0, s_hbm.at[pl.ds(off, CH)])


def _combine(yg, dest1, dest2, g1x, g2x):
    return pl.kernel(
        _combine_body,
        out_type=jax.ShapeDtypeStruct((N, DM), jnp.float32),
        mesh=_SC_MESH,
        scratch_types=[
            pltpu.VMEM((CH, DM), jnp.float32),
            pltpu.VMEM((CH, DM), jnp.float32),
            pltpu.VMEM((CH, 16), jnp.float32),
            pltpu.VMEM((CH, 16), jnp.float32),
            pltpu.VMEM((CH,), jnp.int32),
            pltpu.VMEM((CH,), jnp.int32),
            pltpu.SemaphoreType.DMA,
            pltpu.SemaphoreType.DMA,
        ],
    )(yg, dest1, dest2, g1x, g2x)


def _log_body(s_ref, o_ref):
    o_ref[...] = jnp.log(s_ref[...] + EPS)


def _log_eps(S):
    return pl.pallas_call(
        _log_body,
        grid=(NB,),
        in_specs=[pl.BlockSpec((BT, DM), lambda t: (t, 0))],
        out_specs=pl.BlockSpec((BT, DM), lambda t: (t, 0)),
        out_shape=jax.ShapeDtypeStruct((N, DM), jnp.float32),
    )(S)


def kernel(x, Wg, bg, W1, b1, W2, b2):
    e1, e2, g1, g2, r1, r2, counts = _gate_route(x, Wg, bg)
    d1, d2, te = _make_dest(counts, e1, e2, r1, r2)
    dest1 = d1.reshape(N)
    dest2 = d2.reshape(N)
    tile_expert = te.reshape(BT)[:NTILE]

    xg = _dispatch(x, dest1, dest2)
    yg = _expert_mlp(tile_expert, xg, W1, b1, W2, b2)
    S = _combine(yg, dest1, dest2, g1.reshape(N, 16), g2.reshape(N, 16))
    return _log_eps(S)


# speedup vs baseline: 2.1021x; 1.0161x over previous
import functools

import jax
import jax.numpy as jnp
import numpy as np
from jax import lax
from jax.experimental import pallas as pl
from jax.experimental.pallas import tpu as pltpu
from jax.experimental.pallas import tpu_sc as plsc

N = 4096
DM = 1024
DH = 4096
E = 16
BT = 128
NTILE = (8192 + E * BT) // BT
NP = NTILE * BT
NB = N // BT
EPS = float(np.finfo(float).eps)


def _gate_route_body(x_ref, wg_ref, bg_ref, e1_ref, e2_ref, g1_ref, g2_ref,
                     r1_ref, r2_ref, cnt_ref, run_ref):
    t = pl.program_id(0)

    @pl.when(t == 0)
    def _init():
        run_ref[...] = jnp.zeros_like(run_ref)

    x = x_ref[...]
    logits = lax.dot_general(x, wg_ref[...], (((1,), (1,)), ((), ()))) + bg_ref[...]
    iota = lax.broadcasted_iota(jnp.int32, (BT, E), 1).astype(jnp.float32)
    m1 = jnp.max(logits, axis=1, keepdims=True)
    e1 = jnp.min(jnp.where(logits == m1, iota, float(E)), axis=1, keepdims=True)
    masked = jnp.where(iota == e1, -jnp.inf, logits)
    m2 = jnp.max(masked, axis=1, keepdims=True)
    e2 = jnp.min(jnp.where(masked == m2, iota, float(E)), axis=1, keepdims=True)
    g1 = 1.0 / (1.0 + jnp.exp(m2 - m1))
    g2 = 1.0 - g1

    O1 = (iota == e1).astype(jnp.float32)
    O2 = (iota == e2).astype(jnp.float32)
    C = O1 + O2
    r_i = lax.broadcasted_iota(jnp.int32, (BT, BT), 0)
    c_i = lax.broadcasted_iota(jnp.int32, (BT, BT), 1)
    tri = (c_i < r_i).astype(jnp.float32)
    cex = lax.dot_general(tri, C, (((1,), (0,)), ((), ()))) + run_ref[...]
    r1 = jnp.sum(O1 * cex, axis=1)
    r2 = jnp.sum(O2 * cex, axis=1)

    newrun = run_ref[...] + jnp.sum(C, axis=0, keepdims=True)
    run_ref[...] = newrun
    cnt_ref[...] = newrun

    e1_ref[...] = e1.astype(jnp.int32).reshape(1, 1, BT)
    e2_ref[...] = e2.astype(jnp.int32).reshape(1, 1, BT)
    g1_ref[...] = jnp.broadcast_to(g1, (BT, 16)).reshape(1, BT, 16)
    g2_ref[...] = jnp.broadcast_to(g2, (BT, 16)).reshape(1, BT, 16)
    r1_ref[...] = r1.astype(jnp.int32).reshape(1, 1, BT)
    r2_ref[...] = r2.astype(jnp.int32).reshape(1, 1, BT)


def _gate_route(x, Wg, bg):
    i32b = jax.ShapeDtypeStruct((NB, 1, BT), jnp.int32)
    f32x = jax.ShapeDtypeStruct((NB, BT, 16), jnp.float32)
    outs = (i32b, i32b, f32x, f32x, i32b, i32b,
            jax.ShapeDtypeStruct((1, E), jnp.float32))
    blk3 = pl.BlockSpec((1, 1, BT), lambda t: (t, 0, 0))
    blkx = pl.BlockSpec((1, BT, 16), lambda t: (t, 0, 0))
    return pl.pallas_call(
        _gate_route_body,
        grid=(NB,),
        in_specs=[
            pl.BlockSpec((BT, DM), lambda t: (t, 0)),
            pl.BlockSpec((E, DM), lambda t: (0, 0)),
            pl.BlockSpec((1, E), lambda t: (0, 0)),
        ],
        out_specs=(blk3, blk3, blkx, blkx, blk3, blk3,
                   pl.BlockSpec((1, E), lambda t: (0, 0))),
        out_shape=outs,
        scratch_shapes=[pltpu.VMEM((1, E), jnp.float32)],
        compiler_params=pltpu.CompilerParams(
            dimension_semantics=("arbitrary",)),
    )(x, Wg, bg.reshape(1, E))


def _make_dest_body(cnt_ref, e1_ref, e2_ref, r1_ref, r2_ref,
                    d1_ref, d2_ref, te_ref):
    counts = cnt_ref[...]
    ccnt = jnp.ceil(counts / BT) * BT
    e_i = lax.broadcasted_iota(jnp.int32, (E, E), 0)
    e_j = lax.broadcasted_iota(jnp.int32, (E, E), 1)
    tri = (e_j < e_i).astype(jnp.float32)
    offs = lax.dot_general(ccnt, tri, (((1,), (1,)), ((), ())))

    iota = lax.broadcasted_iota(jnp.int32, (BT, E), 1)
    e1 = e1_ref[...].reshape(BT, 1)
    e2 = e2_ref[...].reshape(BT, 1)
    O1 = (iota == e1).astype(jnp.float32)
    O2 = (iota == e2).astype(jnp.float32)
    b1 = jnp.sum(O1 * offs, axis=1).astype(jnp.int32)
    b2 = jnp.sum(O2 * offs, axis=1).astype(jnp.int32)
    d1_ref[...] = (b1 + r1_ref[...].reshape(BT)).reshape(1, 1, BT)
    d2_ref[...] = (b2 + r2_ref[...].reshape(BT)).reshape(1, 1, BT)

    ends = offs + ccnt
    tid = lax.broadcasted_iota(jnp.int32, (BT, E), 0).astype(jnp.float32) * float(BT)
    full = jnp.sum((tid >= ends).astype(jnp.int32), axis=1)
    te_ref[...] = jnp.minimum(full, E - 1).reshape(1, BT)


def _make_dest(counts, e1, e2, r1, r2):
    i32b = jax.ShapeDtypeStruct((NB, 1, BT), jnp.int32)
    blk3 = pl.BlockSpec((1, 1, BT), lambda t: (t, 0, 0))
    return pl.pallas_call(
        _make_dest_body,
        grid=(NB,),
        in_specs=[pl.BlockSpec((1, E), lambda t: (0, 0)),
                  blk3, blk3, blk3, blk3],
        out_specs=(blk3, blk3, pl.BlockSpec((1, BT), lambda t: (0, 0))),
        out_shape=(i32b, i32b, jax.ShapeDtypeStruct((1, BT), jnp.int32)),
        compiler_params=pltpu.CompilerParams(
            dimension_semantics=("arbitrary",)),
    )(counts, e1, e2, r1, r2)


DHH = DH // 2


def _mlp_a_body(te_ref, xg_ref, w1_ref, b1_ref, w2_ref, acc_ref):
    h = lax.dot_general(xg_ref[...], w1_ref[0], (((1,), (1,)), ((), ())))
    h = jnp.maximum(h + b1_ref[0, 0], 0.0)
    acc_ref[...] = lax.dot_general(h, w2_ref[0], (((1,), (1,)), ((), ())))


def _mlp_b_body(te_ref, xg_ref, acc_ref, w1_ref, b1_ref, w2_ref, b2_ref,
                yg_ref):
    h = lax.dot_general(xg_ref[...], w1_ref[0], (((1,), (1,)), ((), ())))
    h = jnp.maximum(h + b1_ref[0, 0], 0.0)
    y = acc_ref[...] + lax.dot_general(h, w2_ref[0], (((1,), (1,)), ((), ())))
    yg_ref[...] = jnp.exp(jnp.maximum(y + b2_ref[0], 0.0))


def _expert_mlp(tile_expert, xg, W1, b1, W2, b2):
    b1r = b1.reshape(E, 2, 1, DHH)
    gs_a = pltpu.PrefetchScalarGridSpec(
        num_scalar_prefetch=1,
        grid=(NTILE,),
        in_specs=[
            pl.BlockSpec((BT, DM), lambda t, te: (t, 0)),
            pl.BlockSpec((1, DHH, DM), lambda t, te: (te[t], 0, 0)),
            pl.BlockSpec((1, 1, 1, DHH), lambda t, te: (te[t], 0, 0, 0)),
            pl.BlockSpec((1, DM, DHH), lambda t, te: (te[t], 0, 0)),
        ],
        out_specs=pl.BlockSpec((BT, DM), lambda t, te: (t, 0)),
    )
    acc = pl.pallas_call(
        _mlp_a_body,
        grid_spec=gs_a,
        out_shape=jax.ShapeDtypeStruct((NP, DM), jnp.float32),
        compiler_params=pltpu.CompilerParams(
            dimension_semantics=("arbitrary",)),
    )(tile_expert, xg, W1, b1r, W2)
    gs_b = pltpu.PrefetchScalarGridSpec(
        num_scalar_prefetch=1,
        grid=(NTILE,),
        in_specs=[
            pl.BlockSpec((BT, DM), lambda t, te: (t, 0)),
            pl.BlockSpec((BT, DM), lambda t, te: (t, 0)),
            pl.BlockSpec((1, DHH, DM), lambda t, te: (te[t], 1, 0)),
            pl.BlockSpec((1, 1, 1, DHH), lambda t, te: (te[t], 1, 0, 0)),
            pl.BlockSpec((1, DM, DHH), lambda t, te: (te[t], 0, 1)),
            pl.BlockSpec((1, 1, DM), lambda t, te: (te[t], 0, 0)),
        ],
        out_specs=pl.BlockSpec((BT, DM), lambda t, te: (t, 0)),
    )
    return pl.pallas_call(
        _mlp_b_body,
        grid_spec=gs_b,
        out_shape=jax.ShapeDtypeStruct((NP, DM), jnp.float32),
        compiler_params=pltpu.CompilerParams(
            dimension_semantics=("arbitrary",)),
    )(tile_expert, xg, acc, W1, b1r, W2, b2.reshape(E, 1, DM))


_SC_MESH = plsc.VectorSubcoreMesh(core_axis_name="c", subcore_axis_name="s")
_NC = _SC_MESH.num_cores
_NS = _SC_MESH.num_subcores
_NW = _NC * _NS
TPW = N // _NW
CH = 32


def _dispatch_body(x_hbm, d1_hbm, d2_hbm, xg_hbm, buf, idx1, idx2, sem1, sem2):
    wid = lax.axis_index("s") * _NC + lax.axis_index("c")
    base = pl.multiple_of(wid * TPW, TPW)
    for c in range(TPW // CH):
        off = base + c * CH
        pltpu.sync_copy(x_hbm.at[pl.ds(off, CH)], buf)
        pltpu.sync_copy(d1_hbm.at[pl.ds(off, CH)], idx1)
        pltpu.sync_copy(d2_hbm.at[pl.ds(off, CH)], idx2)
        cp1 = pltpu.async_copy(buf, xg_hbm.at[idx1], sem1)
        cp2 = pltpu.async_copy(buf, xg_hbm.at[idx2], sem2)
        cp1.wait()
        cp2.wait()


def _dispatch(x, dest1, dest2):
    return pl.kernel(
        _dispatch_body,
        out_type=jax.ShapeDtypeStruct((NP, DM), jnp.float32),
        mesh=_SC_MESH,
        scratch_types=[
            pltpu.VMEM((CH, DM), jnp.float32),
            pltpu.VMEM((CH,), jnp.int32),
            pltpu.VMEM((CH,), jnp.int32),
            pltpu.SemaphoreType.DMA,
            pltpu.SemaphoreType.DMA,
        ],
    )(x, dest1, dest2)


def _combine_body(yg_hbm, d1_hbm, d2_hbm, g1_hbm, g2_hbm, s_hbm,
                  buf0, buf1, g1v, g2v, idx1, idx2, sem1, sem2):
    wid = lax.axis_index("s") * _NC + lax.axis_index("c")
    base = pl.multiple_of(wid * TPW, TPW)
    for c in range(TPW // CH):
        off = base + c * CH
        pltpu.sync_copy(d1_hbm.at[pl.ds(off, CH)], idx1)
        pltpu.sync_copy(d2_hbm.at[pl.ds(off, CH)], idx2)
        pltpu.sync_copy(g1_hbm.at[pl.ds(off, CH)], g1v)
        pltpu.sync_copy(g2_hbm.at[pl.ds(off, CH)], g2v)
        cp1 = pltpu.async_copy(yg_hbm.at[idx1], buf0, sem1)
        cp2 = pltpu.async_copy(yg_hbm.at[idx2], buf1, sem2)
        cp1.wait()
        cp2.wait()

        def tok(i, _):
            s1 = g1v[i, :]
            s2 = g2v[i, :]

            def vec(v, _):
                sl = pl.ds(v * 16, 16)
                buf0[i, sl] = s1 * buf0[i, sl] + s2 * buf1[i, sl]
                return 0

            lax.fori_loop(0, DM // 16, vec, 0, unroll=4)
            return 0

        lax.fori_loop(0, CH, tok, 0)
        pltpu.sync_copy(buf0, s_hbm.at[pl.ds(off, CH)])


def _combine(yg, dest1, dest2, g1x, g2x):
    return pl.kernel(
        _combine_body,
        out_type=jax.ShapeDtypeStruct((N, DM), jnp.float32),
        mesh=_SC_MESH,
        scratch_types=[
            pltpu.VMEM((CH, DM), jnp.float32),
            pltpu.VMEM((CH, DM), jnp.float32),
            pltpu.VMEM((CH, 16), jnp.float32),
            pltpu.VMEM((CH, 16), jnp.float32),
            pltpu.VMEM((CH,), jnp.int32),
            pltpu.VMEM((CH,), jnp.int32),
            pltpu.SemaphoreType.DMA,
            pltpu.SemaphoreType.DMA,
        ],
    )(yg, dest1, dest2, g1x, g2x)


def _log_body(s_ref, o_ref):
    o_ref[...] = jnp.log(s_ref[...] + EPS)


def _log_eps(S):
    return pl.pallas_call(
        _log_body,
        grid=(NB,),
        in_specs=[pl.BlockSpec((BT, DM), lambda t: (t, 0))],
        out_specs=pl.BlockSpec((BT, DM), lambda t: (t, 0)),
        out_shape=jax.ShapeDtypeStruct((N, DM), jnp.float32),
    )(S)


def kernel(x, Wg, bg, W1, b1, W2, b2):
    e1, e2, g1, g2, r1, r2, counts = _gate_route(x, Wg, bg)
    d1, d2, te = _make_dest(counts, e1, e2, r1, r2)
    dest1 = d1.reshape(N)
    dest2 = d2.reshape(N)
    tile_expert = te.reshape(BT)[:NTILE]

    xg = _dispatch(x, dest1, dest2)
    yg = _expert_mlp(tile_expert, xg, W1, b1, W2, b2)
    S = _combine(yg, dest1, dest2, g1.reshape(N, 16), g2.reshape(N, 16))
    return _log_eps(S)
